# baseline (speedup 1.0000x reference)
import functools

import jax
import jax.numpy as jnp
from jax import lax
from jax.experimental import pallas as pl
from jax.experimental.pallas import tpu as pltpu
from jax.experimental.pallas import tpu_sc as plsc

N_ROWS = 10000
HIDDEN = 256
NUM_SEGS = 64
NUM_CLASSES = 10

NC = 2
NS = 16
L = 16
NW = NC * NS
CHUNK = (N_ROWS // NW) // 8 * 8
TAIL = N_ROWS - CHUNK * NW
TAIL_PER = 8
NCOL = HIDDEN // L


SUB = CHUNK // 3
NSUB = CHUNK // SUB


def _sc_pool_kernel(x_hbm, ids_hbm, sums_hbm, cnts_hbm,
                    buf0_v, buf1_v, ids_v, x_tail_v, ids_tail_v,
                    acc_v, cnt_v, sem0, sem1):
    cid = lax.axis_index("c")
    sid = lax.axis_index("s")
    wid = cid * NS + sid
    base = wid * CHUNK

    zeros16 = jnp.zeros((L,), jnp.float32)
    ones16 = jnp.ones((L,), jnp.float32)
    sixteen16 = jnp.full((L,), float(L), jnp.float32)

    cp0 = pltpu.async_copy(x_hbm.at[pl.ds(base, SUB)], buf0_v, sem0)
    cp1 = pltpu.async_copy(x_hbm.at[pl.ds(base + SUB, SUB)], buf1_v, sem1)
    pltpu.sync_copy(ids_hbm.at[pl.ds(base, CHUNK)], ids_v)

    @pl.loop(0, NUM_SEGS)
    def _zero(r):
        for c in range(NCOL):
            acc_v[r, pl.ds(c * L, L)] = zeros16
        cnt_v[r, :] = zeros16

    def _process_sub(buf, s):
        def _load_row(r):
            return [buf[r, pl.ds(c * L, L)] for c in range(NCOL)]

        def _store_row(seg, vals, count):
            for c in range(NCOL):
                plsc.addupdate(acc_v.at[seg, pl.ds(c * L, L)], vals[c])
            plsc.addupdate(cnt_v.at[seg], count)

        @pl.loop(0, SUB // L)
        def _grp(g):
            ids16 = ids_v[pl.ds(s * SUB + g * L, L)]
            uniform = ids16[0] == ids16[L - 1]

            @pl.when(uniform)
            def _fast():
                accs = _load_row(g * L)
                for j in range(1, L):
                    vals = _load_row(g * L + j)
                    accs_new = [a + v for a, v in zip(accs, vals)]
                    accs[:] = accs_new
                _store_row(ids16[0], accs, sixteen16)

            @pl.when(jnp.logical_not(uniform))
            def _slow():
                for j in range(L):
                    _store_row(ids16[j], _load_row(g * L + j), ones16)

        if SUB % L:
            ids16_t = ids_v[pl.ds(s * SUB + SUB - L, L)]
            for j in range(L - SUB % L, L):
                _store_row(ids16_t[j], _load_row(SUB - L + j), ones16)

    cp0.wait()
    _process_sub(buf0_v, 0)
    cp2 = pltpu.async_copy(x_hbm.at[pl.ds(base + 2 * SUB, SUB)], buf0_v,
                           sem0)
    cp1.wait()
    _process_sub(buf1_v, 1)
    cp2.wait()
    _process_sub(buf0_v, 2)

    @pl.when(wid < TAIL // TAIL_PER)
    def _tail():
        tbase = CHUNK * NW + wid * TAIL_PER
        pltpu.sync_copy(ids_hbm.at[pl.ds(tbase, TAIL_PER)],
                        ids_tail_v.at[pl.ds(0, TAIL_PER)])
        pltpu.sync_copy(x_hbm.at[pl.ds(tbase, TAIL_PER)], x_tail_v)

        ids16_x = ids_tail_v[...]
        for j in range(TAIL_PER):
            seg = ids16_x[j]
            for c in range(NCOL):
                plsc.addupdate(acc_v.at[seg, pl.ds(c * L, L)],
                               x_tail_v[j, pl.ds(c * L, L)])
            plsc.addupdate(cnt_v.at[seg], ones16)

    pltpu.sync_copy(acc_v, sums_hbm.at[wid])
    pltpu.sync_copy(cnt_v, cnts_hbm.at[wid])


def _mlp_head_kernel(sums_ref, cnts_ref, w1_ref, b1_ref, w2_ref, b2_ref,
                     out_ref):
    sums = jnp.sum(sums_ref[...], axis=0)
    cnts = jnp.sum(cnts_ref[...], axis=0)[:, :1]
    pool = sums / jnp.maximum(cnts, 1.0)
    h = jax.lax.dot(pool, w1_ref[...],
                    precision=jax.lax.Precision.HIGHEST,
                    preferred_element_type=jnp.float32)
    h = jnp.maximum(h + b1_ref[...], 0.0)
    logits = jax.lax.dot(h, w2_ref[...],
                         precision=jax.lax.Precision.HIGHEST,
                         preferred_element_type=jnp.float32)
    out_ref[...] = logits + b2_ref[...]


@jax.jit
def _run(x_e, batch_node, W1, b1, W2, b2):
    ids32 = batch_node.astype(jnp.int32)

    sc_pool = pl.kernel(
        _sc_pool_kernel,
        out_type=[
            jax.ShapeDtypeStruct((NW, NUM_SEGS, HIDDEN), jnp.float32),
            jax.ShapeDtypeStruct((NW, NUM_SEGS, L), jnp.float32),
        ],
        mesh=plsc.VectorSubcoreMesh(core_axis_name="c", subcore_axis_name="s"),
        scratch_types=[
            pltpu.VMEM((SUB, HIDDEN), jnp.float32),
            pltpu.VMEM((SUB, HIDDEN), jnp.float32),
            pltpu.VMEM((CHUNK,), jnp.int32),
            pltpu.VMEM((TAIL_PER, HIDDEN), jnp.float32),
            pltpu.VMEM((L,), jnp.int32),
            pltpu.VMEM((NUM_SEGS, HIDDEN), jnp.float32),
            pltpu.VMEM((NUM_SEGS, L), jnp.float32),
            pltpu.SemaphoreType.DMA,
            pltpu.SemaphoreType.DMA,
        ],
    )
    sums, cnts = sc_pool(x_e, ids32)

    b1r = b1.reshape(1, HIDDEN // 2)
    b2r = b2.reshape(1, NUM_CLASSES)
    logits = pl.pallas_call(
        _mlp_head_kernel,
        out_shape=jax.ShapeDtypeStruct((NUM_SEGS, NUM_CLASSES), jnp.float32),
    )(sums, cnts, W1, b1r, W2, b2r)
    return logits


def kernel(x_e, pos_e, edge_index_e, edge_attr_e, batch_node, batch_edge,
           W1, b1, W2, b2):
    return _run(x_e, batch_node, W1, b1, W2, b2)


# speedup vs baseline: 1.2487x; 1.2487x over previous
import functools

import jax
import jax.numpy as jnp
from jax import lax
from jax.experimental import pallas as pl
from jax.experimental.pallas import tpu as pltpu
from jax.experimental.pallas import tpu_sc as plsc

N_ROWS = 10000
HIDDEN = 256
NUM_SEGS = 64
NUM_CLASSES = 10

NC = 2
NS = 16
L = 16
NW = NC * NS
CHUNK = (N_ROWS // NW) // 8 * 8
TAIL = N_ROWS - CHUNK * NW
TAIL_PER = 8
NCOL = HIDDEN // L


SUB = CHUNK // 3
NSUB = CHUNK // SUB


def _sc_pool_kernel(x_hbm, ids_hbm, sums_hbm, cnts_hbm,
                    buf_v, ids_v, x_tail_v, ids_tail_v,
                    acc_v, cnt_v, sem0, sem1):
    cid = lax.axis_index("c")
    sid = lax.axis_index("s")
    wid = cid * NS + sid
    base = wid * CHUNK

    zeros16 = jnp.zeros((L,), jnp.float32)
    ones16 = jnp.ones((L,), jnp.float32)
    sixteen16 = jnp.full((L,), float(L), jnp.float32)

    pltpu.async_copy(x_hbm.at[pl.ds(base, SUB)], buf_v.at[0], sem0)
    pltpu.async_copy(x_hbm.at[pl.ds(base + SUB, SUB)], buf_v.at[1], sem1)
    pltpu.sync_copy(ids_hbm.at[pl.ds(base, CHUNK)], ids_v)

    @pl.loop(0, NUM_SEGS)
    def _zero(r):
        for c in range(NCOL):
            acc_v[r, pl.ds(c * L, L)] = zeros16
        cnt_v[r, :] = zeros16

    @pl.loop(0, NSUB)
    def _sub(s):
        b = lax.rem(s, 2)

        @pl.when(b == 0)
        def _w0():
            pltpu.make_async_copy(
                x_hbm.at[pl.ds(base, SUB)], buf_v.at[0], sem0).wait()

        @pl.when(b != 0)
        def _w1():
            pltpu.make_async_copy(
                x_hbm.at[pl.ds(base, SUB)], buf_v.at[1], sem1).wait()

        def _load_row(r):
            return [buf_v[b, r, pl.ds(c * L, L)] for c in range(NCOL)]

        def _store_row(seg, vals, count):
            for c in range(NCOL):
                plsc.addupdate(acc_v.at[seg, pl.ds(c * L, L)], vals[c])
            plsc.addupdate(cnt_v.at[seg], count)

        @pl.loop(0, SUB // L)
        def _grp(g):
            ids16 = ids_v[pl.ds(s * SUB + g * L, L)]
            uniform = ids16[0] == ids16[L - 1]

            @pl.when(uniform)
            def _fast():
                accs = _load_row(g * L)
                for j in range(1, L):
                    vals = _load_row(g * L + j)
                    accs_new = [a + v for a, v in zip(accs, vals)]
                    accs[:] = accs_new
                _store_row(ids16[0], accs, sixteen16)

            @pl.when(jnp.logical_not(uniform))
            def _slow():
                for j in range(L):
                    _store_row(ids16[j], _load_row(g * L + j), ones16)

        if SUB % L:
            ids16_t = ids_v[pl.ds(s * SUB + SUB - L, L)]
            for j in range(L - SUB % L, L):
                _store_row(ids16_t[j], _load_row(SUB - L + j), ones16)

        @pl.when(s + 2 < NSUB)
        def _next():
            nbase = base + (s + 2) * SUB

            @pl.when(b == 0)
            def _n0():
                pltpu.async_copy(
                    x_hbm.at[pl.ds(nbase, SUB)], buf_v.at[0], sem0)

            @pl.when(b != 0)
            def _n1():
                pltpu.async_copy(
                    x_hbm.at[pl.ds(nbase, SUB)], buf_v.at[1], sem1)

    @pl.when(wid < TAIL // TAIL_PER)
    def _tail():
        tbase = CHUNK * NW + wid * TAIL_PER
        pltpu.sync_copy(ids_hbm.at[pl.ds(tbase, TAIL_PER)],
                        ids_tail_v.at[pl.ds(0, TAIL_PER)])
        pltpu.sync_copy(x_hbm.at[pl.ds(tbase, TAIL_PER)], x_tail_v)

        ids16_x = ids_tail_v[...]
        for j in range(TAIL_PER):
            seg = ids16_x[j]
            for c in range(NCOL):
                plsc.addupdate(acc_v.at[seg, pl.ds(c * L, L)],
                               x_tail_v[j, pl.ds(c * L, L)])
            plsc.addupdate(cnt_v.at[seg], ones16)

    pltpu.sync_copy(acc_v, sums_hbm.at[wid])
    pltpu.sync_copy(cnt_v, cnts_hbm.at[wid])


def _mlp_head_kernel(sums_ref, cnts_ref, w1_ref, b1_ref, w2_ref, b2_ref,
                     out_ref):
    sums = jnp.sum(sums_ref[...], axis=0)
    cnts = jnp.sum(cnts_ref[...], axis=0)[:, :1]
    pool = sums / jnp.maximum(cnts, 1.0)
    h = jax.lax.dot(pool, w1_ref[...],
                    precision=jax.lax.Precision.HIGHEST,
                    preferred_element_type=jnp.float32)
    h = jnp.maximum(h + b1_ref[...], 0.0)
    logits = jax.lax.dot(h, w2_ref[...],
                         precision=jax.lax.Precision.HIGHEST,
                         preferred_element_type=jnp.float32)
    out_ref[...] = logits + b2_ref[...]


@jax.jit
def _run(x_e, batch_node, W1, b1, W2, b2):
    ids32 = batch_node.astype(jnp.int32)

    sc_pool = pl.kernel(
        _sc_pool_kernel,
        out_type=[
            jax.ShapeDtypeStruct((NW, NUM_SEGS, HIDDEN), jnp.float32),
            jax.ShapeDtypeStruct((NW, NUM_SEGS, L), jnp.float32),
        ],
        mesh=plsc.VectorSubcoreMesh(core_axis_name="c", subcore_axis_name="s"),
        scratch_types=[
            pltpu.VMEM((2, SUB, HIDDEN), jnp.float32),
            pltpu.VMEM((CHUNK,), jnp.int32),
            pltpu.VMEM((TAIL_PER, HIDDEN), jnp.float32),
            pltpu.VMEM((L,), jnp.int32),
            pltpu.VMEM((NUM_SEGS, HIDDEN), jnp.float32),
            pltpu.VMEM((NUM_SEGS, L), jnp.float32),
            pltpu.SemaphoreType.DMA,
            pltpu.SemaphoreType.DMA,
        ],
    )
    sums, cnts = sc_pool(x_e, ids32)

    b1r = b1.reshape(1, HIDDEN // 2)
    b2r = b2.reshape(1, NUM_CLASSES)
    logits = pl.pallas_call(
        _mlp_head_kernel,
        out_shape=jax.ShapeDtypeStruct((NUM_SEGS, NUM_CLASSES), jnp.float32),
    )(sums, cnts, W1, b1r, W2, b2r)
    return logits


def kernel(x_e, pos_e, edge_index_e, edge_attr_e, batch_node, batch_edge,
           W1, b1, W2, b2):
    return _run(x_e, batch_node, W1, b1, W2, b2)


# speedup vs baseline: 1.2539x; 1.0041x over previous
import functools

import jax
import jax.numpy as jnp
from jax import lax
from jax.experimental import pallas as pl
from jax.experimental.pallas import tpu as pltpu
from jax.experimental.pallas import tpu_sc as plsc

N_ROWS = 10000
HIDDEN = 256
NUM_SEGS = 64
NUM_CLASSES = 10

NC = 2
NS = 16
L = 16
NW = NC * NS
CHUNK = (N_ROWS // NW) // 8 * 8
TAIL = N_ROWS - CHUNK * NW
TAIL_PER = 8
NCOL = HIDDEN // L


SUB = CHUNK // 3
NSUB = CHUNK // SUB


def _sc_pool_kernel(x_hbm, ids_hbm, sums_hbm, cnts_hbm,
                    buf_v, ids_v, x_tail_v, ids_tail_v,
                    acc_v, cnt_v, sem0, sem1):
    cid = lax.axis_index("c")
    sid = lax.axis_index("s")
    wid = cid * NS + sid
    base = wid * CHUNK

    zeros16 = jnp.zeros((L,), jnp.float32)
    ones16 = jnp.ones((L,), jnp.float32)
    sixteen16 = jnp.full((L,), float(L), jnp.float32)

    pltpu.async_copy(x_hbm.at[pl.ds(base, SUB)], buf_v.at[0], sem0)
    pltpu.async_copy(x_hbm.at[pl.ds(base + SUB, SUB)], buf_v.at[1], sem1)
    pltpu.sync_copy(ids_hbm.at[pl.ds(base, CHUNK)], ids_v)

    @pl.loop(0, NUM_SEGS)
    def _zero(r):
        for c in range(NCOL):
            acc_v[r, pl.ds(c * L, L)] = zeros16
        cnt_v[r, :] = zeros16

    @pl.loop(0, NSUB)
    def _sub(s):
        b = lax.rem(s, 2)

        @pl.when(b == 0)
        def _w0():
            pltpu.make_async_copy(
                x_hbm.at[pl.ds(base, SUB)], buf_v.at[0], sem0).wait()

        @pl.when(b != 0)
        def _w1():
            pltpu.make_async_copy(
                x_hbm.at[pl.ds(base, SUB)], buf_v.at[1], sem1).wait()

        def _load_row(r):
            return [buf_v[b, r, pl.ds(c * L, L)] for c in range(NCOL)]

        def _store_row(seg, vals, count):
            for c in range(NCOL):
                plsc.addupdate(acc_v.at[seg, pl.ds(c * L, L)], vals[c])
            plsc.addupdate(cnt_v.at[seg], count)

        @pl.loop(0, SUB // L)
        def _grp(g):
            ids16 = ids_v[pl.ds(s * SUB + g * L, L)]
            uniform = ids16[0] == ids16[L - 1]

            @pl.when(uniform)
            def _fast():
                accs = _load_row(g * L)
                prev = _load_row(g * L + 1)
                for j in range(2, L):
                    nxt = []
                    for c in range(NCOL):
                        nxt.append(buf_v[b, g * L + j, pl.ds(c * L, L)])
                        accs[c] = accs[c] + prev[c]
                    prev = nxt
                for c in range(NCOL):
                    accs[c] = accs[c] + prev[c]
                _store_row(ids16[0], accs, sixteen16)

            @pl.when(jnp.logical_not(uniform))
            def _slow():
                for j in range(L):
                    _store_row(ids16[j], _load_row(g * L + j), ones16)

        if SUB % L:
            ids16_t = ids_v[pl.ds(s * SUB + SUB - L, L)]
            for j in range(L - SUB % L, L):
                _store_row(ids16_t[j], _load_row(SUB - L + j), ones16)

        @pl.when(s + 2 < NSUB)
        def _next():
            nbase = base + (s + 2) * SUB

            @pl.when(b == 0)
            def _n0():
                pltpu.async_copy(
                    x_hbm.at[pl.ds(nbase, SUB)], buf_v.at[0], sem0)

            @pl.when(b != 0)
            def _n1():
                pltpu.async_copy(
                    x_hbm.at[pl.ds(nbase, SUB)], buf_v.at[1], sem1)

    @pl.when(sid == 0)
    def _tail():
        tbase = CHUNK * NW + cid * TAIL_PER
        pltpu.sync_copy(ids_hbm.at[pl.ds(tbase, TAIL_PER)],
                        ids_tail_v.at[pl.ds(0, TAIL_PER)])
        pltpu.sync_copy(x_hbm.at[pl.ds(tbase, TAIL_PER)], x_tail_v)

        ids16_x = ids_tail_v[...]
        for j in range(TAIL_PER):
            seg = ids16_x[j]
            for c in range(NCOL):
                plsc.addupdate(acc_v.at[seg, pl.ds(c * L, L)],
                               x_tail_v[j, pl.ds(c * L, L)])
            plsc.addupdate(cnt_v.at[seg], ones16)

    pltpu.sync_copy(acc_v, sums_hbm.at[wid])
    pltpu.sync_copy(cnt_v, cnts_hbm.at[wid])


def _mlp_head_kernel(sums_ref, cnts_ref, w1_ref, b1_ref, w2_ref, b2_ref,
                     out_ref):
    sums = jnp.sum(sums_ref[...], axis=0)
    cnts = jnp.sum(cnts_ref[...], axis=0)[:, :1]
    pool = sums / jnp.maximum(cnts, 1.0)
    h = jax.lax.dot(pool, w1_ref[...],
                    precision=jax.lax.Precision.HIGHEST,
                    preferred_element_type=jnp.float32)
    h = jnp.maximum(h + b1_ref[...], 0.0)
    logits = jax.lax.dot(h, w2_ref[...],
                         precision=jax.lax.Precision.HIGHEST,
                         preferred_element_type=jnp.float32)
    out_ref[...] = logits + b2_ref[...]


@jax.jit
def _run(x_e, batch_node, W1, b1, W2, b2):
    ids32 = batch_node.astype(jnp.int32)

    sc_pool = pl.kernel(
        _sc_pool_kernel,
        out_type=[
            jax.ShapeDtypeStruct((NW, NUM_SEGS, HIDDEN), jnp.float32),
            jax.ShapeDtypeStruct((NW, NUM_SEGS, L), jnp.float32),
        ],
        mesh=plsc.VectorSubcoreMesh(core_axis_name="c", subcore_axis_name="s"),
        scratch_types=[
            pltpu.VMEM((2, SUB, HIDDEN), jnp.float32),
            pltpu.VMEM((CHUNK,), jnp.int32),
            pltpu.VMEM((TAIL_PER, HIDDEN), jnp.float32),
            pltpu.VMEM((L,), jnp.int32),
            pltpu.VMEM((NUM_SEGS, HIDDEN), jnp.float32),
            pltpu.VMEM((NUM_SEGS, L), jnp.float32),
            pltpu.SemaphoreType.DMA,
            pltpu.SemaphoreType.DMA,
        ],
    )
    sums, cnts = sc_pool(x_e, ids32)

    b1r = b1.reshape(1, HIDDEN // 2)
    b2r = b2.reshape(1, NUM_CLASSES)
    logits = pl.pallas_call(
        _mlp_head_kernel,
        out_shape=jax.ShapeDtypeStruct((NUM_SEGS, NUM_CLASSES), jnp.float32),
    )(sums, cnts, W1, b1r, W2, b2r)
    return logits


def kernel(x_e, pos_e, edge_index_e, edge_attr_e, batch_node, batch_edge,
           W1, b1, W2, b2):
    return _run(x_e, batch_node, W1, b1, W2, b2)


# speedup vs baseline: 1.2556x; 1.0014x over previous
import jax
import jax.numpy as jnp
from jax import lax
from jax.experimental import pallas as pl
from jax.experimental.pallas import tpu as pltpu
from jax.experimental.pallas import tpu_sc as plsc

N_ROWS = 10000
HIDDEN = 256
NUM_SEGS = 64
NUM_CLASSES = 10

NC = 2
NS = 16
L = 16
NW = NC * NS
CHUNK = (N_ROWS // NW) // 8 * 8
TAIL_PER = 8
NCOL = HIDDEN // L

SUB = CHUNK // 3
NSUB = CHUNK // SUB


def _sc_pool_kernel(x_hbm, ids_hbm, sums_hbm, cnts_hbm,
                    buf_v, ids_v, x_tail_v, ids_tail_v,
                    acc_v, cnt_v, sem0, sem1):
    cid = lax.axis_index("c")
    sid = lax.axis_index("s")
    wid = cid * NS + sid
    base = wid * CHUNK

    zeros16 = jnp.zeros((L,), jnp.float32)
    ones16 = jnp.ones((L,), jnp.float32)
    sixteen16 = jnp.full((L,), float(L), jnp.float32)

    pltpu.async_copy(x_hbm.at[pl.ds(base, SUB)], buf_v.at[0], sem0)
    pltpu.async_copy(x_hbm.at[pl.ds(base + SUB, SUB)], buf_v.at[1], sem1)
    pltpu.sync_copy(ids_hbm.at[pl.ds(base, CHUNK)], ids_v)

    @pl.loop(0, NUM_SEGS)
    def _zero(r):
        for c in range(NCOL):
            acc_v[r, pl.ds(c * L, L)] = zeros16
        cnt_v[r, :] = zeros16

    @pl.loop(0, NSUB)
    def _sub(s):
        b = lax.rem(s, 2)

        @pl.when(b == 0)
        def _w0():
            pltpu.make_async_copy(
                x_hbm.at[pl.ds(base, SUB)], buf_v.at[0], sem0).wait()

        @pl.when(b != 0)
        def _w1():
            pltpu.make_async_copy(
                x_hbm.at[pl.ds(base, SUB)], buf_v.at[1], sem1).wait()

        def _load_row(r):
            return [buf_v[b, r, pl.ds(c * L, L)] for c in range(NCOL)]

        def _store_row(seg, vals, count):
            for c in range(NCOL):
                plsc.addupdate(acc_v.at[seg, pl.ds(c * L, L)], vals[c])
            plsc.addupdate(cnt_v.at[seg], count)

        @pl.loop(0, SUB // L)
        def _grp(g):
            ids16 = ids_v[pl.ds(s * SUB + g * L, L)]
            uniform = ids16[0] == ids16[L - 1]

            @pl.when(uniform)
            def _fast():
                accs = _load_row(g * L)
                prev = _load_row(g * L + 1)
                for j in range(2, L):
                    nxt = []
                    for c in range(NCOL):
                        nxt.append(buf_v[b, g * L + j, pl.ds(c * L, L)])
                        accs[c] = accs[c] + prev[c]
                    prev = nxt
                for c in range(NCOL):
                    accs[c] = accs[c] + prev[c]
                _store_row(ids16[0], accs, sixteen16)

            @pl.when(jnp.logical_not(uniform))
            def _slow():
                for j in range(L):
                    _store_row(ids16[j], _load_row(g * L + j), ones16)

        if SUB % L:
            ids16_t = ids_v[pl.ds(s * SUB + SUB - L, L)]
            for j in range(L - SUB % L, L):
                _store_row(ids16_t[j], _load_row(SUB - L + j), ones16)

        @pl.when(s + 2 < NSUB)
        def _next():
            nbase = base + (s + 2) * SUB

            @pl.when(b == 0)
            def _n0():
                pltpu.async_copy(
                    x_hbm.at[pl.ds(nbase, SUB)], buf_v.at[0], sem0)

            @pl.when(b != 0)
            def _n1():
                pltpu.async_copy(
                    x_hbm.at[pl.ds(nbase, SUB)], buf_v.at[1], sem1)

    @pl.when(sid == 0)
    def _tail():
        tbase = CHUNK * NW + cid * TAIL_PER
        pltpu.sync_copy(ids_hbm.at[pl.ds(tbase, TAIL_PER)],
                        ids_tail_v.at[pl.ds(0, TAIL_PER)])
        pltpu.sync_copy(x_hbm.at[pl.ds(tbase, TAIL_PER)], x_tail_v)

        ids16_x = ids_tail_v[...]
        for j in range(TAIL_PER):
            seg = ids16_x[j]
            for c in range(NCOL):
                plsc.addupdate(acc_v.at[seg, pl.ds(c * L, L)],
                               x_tail_v[j, pl.ds(c * L, L)])
            plsc.addupdate(cnt_v.at[seg], ones16)

    pltpu.sync_copy(acc_v, sums_hbm.at[wid])
    pltpu.sync_copy(cnt_v, cnts_hbm.at[wid])


def _mlp_head_kernel(sums_ref, cnts_ref, w1_ref, b1_ref, w2_ref, b2_ref,
                     out_ref):
    sums = jnp.sum(sums_ref[...], axis=0)
    cnts = jnp.sum(cnts_ref[...], axis=0)[:, :1]
    pool = sums / jnp.maximum(cnts, 1.0)
    h = jax.lax.dot(pool, w1_ref[...],
                    precision=jax.lax.Precision.HIGHEST,
                    preferred_element_type=jnp.float32)
    h = jnp.maximum(h + b1_ref[...], 0.0)
    logits = jax.lax.dot(h, w2_ref[...],
                         precision=jax.lax.Precision.HIGHEST,
                         preferred_element_type=jnp.float32)
    out_ref[...] = logits + b2_ref[...]


@jax.jit
def _run(x_e, batch_node, W1, b1, W2, b2):
    ids32 = batch_node.astype(jnp.int32)

    sc_pool = pl.kernel(
        _sc_pool_kernel,
        out_type=[
            jax.ShapeDtypeStruct((NW, NUM_SEGS, HIDDEN), jnp.float32),
            jax.ShapeDtypeStruct((NW, NUM_SEGS, L), jnp.float32),
        ],
        mesh=plsc.VectorSubcoreMesh(core_axis_name="c", subcore_axis_name="s"),
        scratch_types=[
            pltpu.VMEM((2, SUB, HIDDEN), jnp.float32),
            pltpu.VMEM((CHUNK,), jnp.int32),
            pltpu.VMEM((TAIL_PER, HIDDEN), jnp.float32),
            pltpu.VMEM((L,), jnp.int32),
            pltpu.VMEM((NUM_SEGS, HIDDEN), jnp.float32),
            pltpu.VMEM((NUM_SEGS, L), jnp.float32),
            pltpu.SemaphoreType.DMA,
            pltpu.SemaphoreType.DMA,
        ],
    )
    sums, cnts = sc_pool(x_e, ids32)

    b1r = b1.reshape(1, HIDDEN // 2)
    b2r = b2.reshape(1, NUM_CLASSES)
    logits = pl.pallas_call(
        _mlp_head_kernel,
        out_shape=jax.ShapeDtypeStruct((NUM_SEGS, NUM_CLASSES), jnp.float32),
    )(sums, cnts, W1, b1r, W2, b2r)
    return logits


def kernel(x_e, pos_e, edge_index_e, edge_attr_e, batch_node, batch_edge,
           W1, b1, W2, b2):
    return _run(x_e, batch_node, W1, b1, W2, b2)
